# Initial kernel scaffold; baseline (speedup 1.0000x reference)
import jax
import jax.numpy as jnp
from jax.experimental import pallas as pl


def kernel(ed, threshold_offsets):
    raise NotImplementedError("write your pallas kernel here")


# speedup vs baseline: 1.0064x; 1.0064x over previous
import functools

import numpy as np
import jax
import jax.numpy as jnp
from jax.experimental import pallas as pl
from jax.experimental.pallas import tpu as pltpu

TEMPERATURE = 8.0
OFFSET_SCALE = 0.2
MIN_GAP = 0.001


def _quantile_rank_weights(n: int):
    s = ((np.float32(0.55) + np.float32(0.25)) + np.float32(0.15)) + np.float32(0.05)
    fr = np.array([0.55, 0.25, 0.15, 0.05], dtype=np.float32) / np.maximum(s, np.float32(1e-8))
    cdf = np.cumsum(fr).astype(np.float32)[:3]
    q = (cdf * (np.float32(n) - np.float32(1.0))).astype(np.float32)
    low = np.clip(np.floor(q), 0, n - 1)
    high = np.clip(np.ceil(q), 0, n - 1)
    hw = (q - low).astype(np.float32)
    lw = (np.float32(1.0) - hw).astype(np.float32)
    return low.astype(np.int64), high.astype(np.int64), lw, hw


def _routing_body(qv_ref, off_ref, ed_ref, route_ref, probs_ref, th_ref, r_scratch,
                  *, lw, hw, n_cols):
    base = [qv_ref[c] * lw[c] + qv_ref[3 + c] * hw[c] for c in range(3)]
    raw = [base[c] + off_ref[c] for c in range(3)]
    t1 = raw[0]
    t2 = jnp.maximum(raw[1], t1 + MIN_GAP)
    t3 = jnp.maximum(raw[2], t2 + MIN_GAP)
    left_w = jnp.maximum(t2 - t1, 0.001)
    right_w = jnp.maximum(t3 - t2, 0.001)
    c0 = t1 - left_w
    c1 = (t1 + t2) * 0.5
    c2 = (t2 + t3) * 0.5
    c3 = t3 + right_w

    @pl.when(pl.program_id(0) == 0)
    def _():
        rows = jax.lax.broadcasted_iota(jnp.int32, (4 * n_cols, 4 * n_cols), 0)
        cols = jax.lax.broadcasted_iota(jnp.int32, (4 * n_cols, 4 * n_cols), 1)
        tgt = 4 * (rows % n_cols) + rows // n_cols
        r_scratch[...] = (cols == tgt).astype(jnp.float32)

    ed = ed_ref[...]
    d0 = jnp.abs(ed - c0)
    d1 = jnp.abs(ed - c1)
    d2 = jnp.abs(ed - c2)
    d3 = jnp.abs(ed - c3)
    l0 = -TEMPERATURE * d0
    l1 = -TEMPERATURE * d1
    l2 = -TEMPERATURE * d2
    l3 = -TEMPERATURE * d3
    m = jnp.maximum(jnp.maximum(l0, l1), jnp.maximum(l2, l3))
    e0 = jnp.exp(l0 - m)
    e1 = jnp.exp(l1 - m)
    e2 = jnp.exp(l2 - m)
    e3 = jnp.exp(l3 - m)
    ssum = (e0 + e1) + (e2 + e3)
    p = jnp.concatenate([e0 / ssum, e1 / ssum, e2 / ssum, e3 / ssum], axis=1)
    probs_ref[...] = jnp.dot(p, r_scratch[...], preferred_element_type=jnp.float32)

    i32 = jnp.int32
    route_ref[...] = ((ed > t1).astype(i32) + (ed > t2).astype(i32)
                      + (ed > t3).astype(i32))

    lane = jax.lax.broadcasted_iota(jnp.int32, (1, 8), 1)
    th = jnp.where(lane == 0, t1, jnp.where(lane == 1, t2, jnp.where(lane == 2, t3, 0.0)))
    th_ref[...] = th


def _routing_pass(ed, qvals, offs, lw, hw):
    n_rows, n_cols = ed.shape
    block = 512
    grid = (n_rows // block,)
    route, probs2d, th = pl.pallas_call(
        functools.partial(_routing_body, lw=tuple(np.float32(x) for x in lw),
                          hw=tuple(np.float32(x) for x in hw), n_cols=n_cols),
        grid=grid,
        in_specs=[
            pl.BlockSpec(memory_space=pltpu.SMEM),
            pl.BlockSpec(memory_space=pltpu.SMEM),
            pl.BlockSpec((block, n_cols), lambda i: (i, 0)),
        ],
        out_specs=[
            pl.BlockSpec((block, n_cols), lambda i: (i, 0)),
            pl.BlockSpec((block, 4 * n_cols), lambda i: (i, 0)),
            pl.BlockSpec((1, 8), lambda i: (0, 0)),
        ],
        out_shape=[
            jax.ShapeDtypeStruct((n_rows, n_cols), jnp.int32),
            jax.ShapeDtypeStruct((n_rows, 4 * n_cols), jnp.float32),
            jax.ShapeDtypeStruct((1, 8), jnp.float32),
        ],
        scratch_shapes=[pltpu.VMEM((4 * n_cols, 4 * n_cols), jnp.float32)],
    )(qvals, offs, ed)
    probs = probs2d.reshape(n_rows, n_cols, 4)
    thresholds = th.reshape(8)[:3]
    return route, probs, thresholds


def kernel(ed, threshold_offsets):
    n = ed.size
    low, high, lw, hw = _quantile_rank_weights(n)

    flat = ed.reshape(-1)
    srt = jnp.sort(flat)
    vlo = srt[jnp.array(low)]
    vhi = srt[jnp.array(high)]
    qvals = jnp.concatenate([vlo, vhi, jnp.zeros((2,), jnp.float32)])

    offs = OFFSET_SCALE * jnp.tanh(threshold_offsets)
    offs = jnp.concatenate([offs, jnp.zeros((1,), jnp.float32)])

    route, probs, thresholds = _routing_pass(ed, qvals, offs, lw, hw)
    return route, probs, thresholds


# speedup vs baseline: 6.4434x; 6.4024x over previous
import functools

import numpy as np
import jax
import jax.numpy as jnp
from jax import lax
from jax.experimental import pallas as pl
from jax.experimental.pallas import tpu as pltpu
from jax.experimental.pallas import tpu_sc as plsc

TEMPERATURE = 8.0
OFFSET_SCALE = 0.2
MIN_GAP = 0.001

_I32MAX = np.int32(2147483647)
_MASK31 = np.int32(0x7FFFFFFF)


def _quantile_rank_weights(n: int):
    s = ((np.float32(0.55) + np.float32(0.25)) + np.float32(0.15)) + np.float32(0.05)
    fr = np.array([0.55, 0.25, 0.15, 0.05], dtype=np.float32) / np.maximum(s, np.float32(1e-8))
    cdf = np.cumsum(fr).astype(np.float32)[:3]
    q = (cdf * (np.float32(n) - np.float32(1.0))).astype(np.float32)
    low = np.clip(np.floor(q), 0, n - 1)
    high = np.clip(np.ceil(q), 0, n - 1)
    hw = (q - low).astype(np.float32)
    lw = (np.float32(1.0) - hw).astype(np.float32)
    return low.astype(np.int64), high.astype(np.int64), lw, hw


def _sortable_key(x):
    b = plsc.bitcast(x, jnp.int32)
    m = lax.shift_right_arithmetic(b, 31)
    return b ^ (m & _MASK31)


def _sc_select(ed_flat, ranks):
    n = ed_flat.shape[0]
    NT = 16
    per_tile = n // NT
    CH = 25600
    NCH = per_tile // CH
    assert per_tile % CH == 0 and CH % 16 == 0
    k0, k1, k2 = (jnp.int32(r) for r in ranks)

    mesh = plsc.VectorSubcoreMesh(core_axis_name="c", subcore_axis_name="s",
                                  num_cores=1)

    def merge_slabs(slab, acc, tmp, rows, halves):
        pltpu.sync_copy(slab.at[0], acc)

        def mbody(t, _):
            pltpu.sync_copy(slab.at[t], tmp)

            def rbody(rr, _):
                for hh in range(halves):
                    acc[rr, pl.ds(hh * 16, 16)] = (
                        acc[rr, pl.ds(hh * 16, 16)] + tmp[rr, pl.ds(hh * 16, 16)])
                return 0

            lax.fori_loop(0, rows, rbody, 0)
            return 0

        lax.fori_loop(1, NT, mbody, 0)

    def scan_hist(buf, rows, halves, kt):
        init = (jnp.zeros((16,), jnp.int32), jnp.zeros((16,), jnp.int32),
                jnp.zeros((16,), jnp.int32))

        def body(r, carry):
            cum, bc, bs = carry
            for h in range(halves):
                v = buf[r, pl.ds(h * 16, 16)]
                s = plsc.cumsum(v)
                cv = cum + s
                mlt = cv <= kt
                bc = bc + jnp.where(mlt, jnp.int32(1), jnp.int32(0))
                bs = bs + jnp.where(mlt, v, jnp.int32(0))
                cum = cum + jnp.max(s)
            return cum, bc, bs

        cum, bc, bs = lax.fori_loop(0, rows, body, init)
        return jnp.sum(bc), jnp.sum(bs)

    def zero2d(buf, rows, halves):
        z = jnp.zeros((16,), jnp.int32)

        def body(r, _):
            for h in range(halves):
                buf[r, pl.ds(h * 16, 16)] = z
            return 0

        lax.fori_loop(0, rows, body, 0)

    @functools.partial(
        pl.kernel,
        out_type=jax.ShapeDtypeStruct((16,), jnp.float32),
        mesh=mesh,
        compiler_params=pltpu.CompilerParams(needs_layout_passes=False),
        scratch_types=[
            pltpu.VMEM((CH,), jnp.float32),
            pltpu.VMEM((128, 32), jnp.int32),
            pltpu.VMEM((128, 32), jnp.int32),
            pltpu.VMEM((128, 32), jnp.int32),
            pltpu.VMEM((128, 32), jnp.int32),
            pltpu.VMEM((16, 16), jnp.int32),
            pltpu.VMEM((16, 16), jnp.int32),
            pltpu.VMEM((16, 16), jnp.int32),
            pltpu.VMEM((128, 32), jnp.int32),
            pltpu.VMEM((16, 16), jnp.int32),
            pltpu.VMEM((3, 16), jnp.int32),
            pltpu.VMEM((16,), jnp.int32),
            pltpu.VMEM((16,), jnp.float32),
            pltpu.HBM((NT, 128, 32), jnp.int32),
            pltpu.HBM((NT, 128, 32), jnp.int32),
            pltpu.HBM((NT, 128, 32), jnp.int32),
            pltpu.HBM((NT, 128, 32), jnp.int32),
            pltpu.HBM((NT, 16, 16), jnp.int32),
            pltpu.HBM((NT, 16, 16), jnp.int32),
            pltpu.HBM((NT, 16, 16), jnp.int32),
            pltpu.HBM((16, 16), jnp.int32),
            pltpu.HBM((3, 16), jnp.int32),
        ],
    )
    def sel_kernel(ed_hbm, out_hbm, dbuf, h1, h2_0, h2_1, h2_2, h3_0, h3_1, h3_2,
                   tmp, minbuf, res_v, rowv, outv,
                   hb1, hb2_0, hb2_1, hb2_2, hb3_0, hb3_1, hb3_2, hbmin, hres):
        sid = lax.axis_index("s")
        base = sid * per_tile
        iota16 = lax.iota(jnp.int32, 16)
        h2s = (h2_0, h2_1, h2_2)
        hb2s = (hb2_0, hb2_1, hb2_2)
        h3s = (h3_0, h3_1, h3_2)
        hb3s = (hb3_0, hb3_1, hb3_2)
        tmp16 = minbuf
        ks = (k0, k1, k2)

        def sweep_data(elem_fn):
            def chunk_body(c, _):
                pltpu.sync_copy(ed_hbm.at[pl.ds(base + c * CH, CH)], dbuf)

                def elem_body(i, _):
                    x = dbuf[pl.ds(i * 16, 16)]
                    elem_fn(_sortable_key(x))
                    return 0

                lax.fori_loop(0, CH // 16, elem_body, 0)
                return 0

            lax.fori_loop(0, NCH, chunk_body, 0)

        def publish(r, a, b):
            vec = jnp.where(iota16 == 0, a, jnp.where(iota16 == 1, b, jnp.int32(0)))
            rowv[...] = vec
            pltpu.sync_copy(rowv, hres.at[r])

        zero2d(h1, 128, 2)

        ones16 = jnp.full((16,), 1, jnp.int32)

        def p1(S):
            b = lax.shift_right_arithmetic(S, 20) + jnp.int32(2048)
            plsc.addupdate_scatter(
                h1, [lax.shift_right_logical(b, 5), b & jnp.int32(31)], ones16)

        sweep_data(p1)
        pltpu.sync_copy(h1, hb1.at[sid])
        plsc.subcore_barrier()
        for r in range(3):
            @pl.when(sid == r)
            def _(r=r):
                merge_slabs(hb1, h1, tmp, 128, 2)
                b, e = scan_hist(h1, 128, 2, ks[r])
                publish(r, b - jnp.int32(2048), ks[r] - e)
        plsc.subcore_barrier()
        pltpu.sync_copy(hres, res_v)
        bin1 = [res_v[r, :][0] for r in range(3)]
        g2 = [res_v[r, :][1] for r in range(3)]

        for r in range(3):
            zero2d(h2s[r], 128, 2)

        def p2(S):
            top = lax.shift_right_arithmetic(S, 20)
            mid = lax.shift_right_arithmetic(S, 8) & jnp.int32(0xFFF)
            row = lax.shift_right_logical(mid, 5)
            col = mid & jnp.int32(31)
            for r in range(3):
                match = top == bin1[r]
                plsc.addupdate_scatter(h2s[r], [row, col], ones16, mask=match)

        sweep_data(p2)
        for r in range(3):
            pltpu.sync_copy(h2s[r], hb2s[r].at[sid])
        plsc.subcore_barrier()
        for r in range(3):
            @pl.when(sid == r)
            def _(r=r):
                merge_slabs(hb2s[r], h2s[r], tmp, 128, 2)
                b, e = scan_hist(h2s[r], 128, 2, g2[r])
                publish(r, lax.shift_left(bin1[r], 12) | b, g2[r] - e)
        plsc.subcore_barrier()
        pltpu.sync_copy(hres, res_v)
        pref24 = [res_v[r, :][0] for r in range(3)]
        g3 = [res_v[r, :][1] for r in range(3)]

        for r in range(3):
            zero2d(h3s[r], 16, 1)

        def p3(S):
            top24 = lax.shift_right_arithmetic(S, 8)
            lowb = S & jnp.int32(0xFF)
            row = lax.shift_right_logical(lowb, 4)
            col = lowb & jnp.int32(15)
            for r in range(3):
                match = top24 == pref24[r]
                plsc.addupdate_scatter(h3s[r], [row, col], ones16, mask=match)

        sweep_data(p3)
        for r in range(3):
            pltpu.sync_copy(h3s[r], hb3s[r].at[sid])
        plsc.subcore_barrier()
        for r in range(3):
            @pl.when(sid == r)
            def _(r=r):
                merge_slabs(hb3s[r], h3s[r], tmp16, 16, 1)
                b, e = scan_hist(h3s[r], 16, 1, g3[r])
                g4 = g3[r] - e
                zeros16 = jnp.zeros((16,), jnp.int32)
                cnt_vec = plsc.load_gather(
                    h3s[r], [zeros16 + lax.shift_right_logical(b, 4),
                             zeros16 + (b & jnp.int32(15))])
                dup = (g4 + jnp.int32(1) < jnp.max(cnt_vec)).astype(jnp.int32)
                publish(r, lax.shift_left(pref24[r], 8) | b, dup)
        plsc.subcore_barrier()
        pltpu.sync_copy(hres, res_v)
        kkey = [res_v[r, :][0] for r in range(3)]
        dup_hi = [res_v[r, :][1] for r in range(3)]

        def chunk_body(c, carry):
            m0, m1, m2 = carry
            pltpu.sync_copy(ed_hbm.at[pl.ds(base + c * CH, CH)], dbuf)

            def elem_body(i, mcar):
                n0, n1, n2 = mcar
                S = _sortable_key(dbuf[pl.ds(i * 16, 16)])
                n0 = jnp.minimum(n0, jnp.where(S > kkey[0], S, _I32MAX))
                n1 = jnp.minimum(n1, jnp.where(S > kkey[1], S, _I32MAX))
                n2 = jnp.minimum(n2, jnp.where(S > kkey[2], S, _I32MAX))
                return n0, n1, n2

            return lax.fori_loop(0, CH // 16, elem_body, (m0, m1, m2))

        init = tuple(jnp.full((16,), 2147483647, jnp.int32) for _ in range(3))
        m0, m1, m2 = lax.fori_loop(0, NCH, chunk_body, init)
        mv = jnp.where(iota16 == 0, jnp.min(m0),
                       jnp.where(iota16 == 1, jnp.min(m1),
                                 jnp.where(iota16 == 2, jnp.min(m2), _I32MAX)))
        rowv[...] = mv
        pltpu.sync_copy(rowv, hbmin.at[sid])
        plsc.subcore_barrier()

        @pl.when(sid == 0)
        def _():
            pltpu.sync_copy(hbmin, minbuf)
            acc = minbuf[0, :]
            for t in range(1, 16):
                acc = jnp.minimum(acc, minbuf[t, :])
            vec = jnp.zeros((16,), jnp.float32)
            for r in range(3):
                klo = kkey[r]
                khi = jnp.where(dup_hi[r] > 0, klo, acc[r])
                blo = jnp.where(klo >= 0, klo, klo ^ _MASK31)
                bhi = jnp.where(khi >= 0, khi, khi ^ _MASK31)
                vlo = lax.bitcast_convert_type(blo, jnp.float32)
                vhi = lax.bitcast_convert_type(bhi, jnp.float32)
                vec = jnp.where(iota16 == r, vlo, vec)
                vec = jnp.where(iota16 == r + 3, vhi, vec)
            outv[...] = vec
            pltpu.sync_copy(outv, out_hbm)

    return sel_kernel(ed_flat)


def _routing_body(qv_ref, off_ref, ed_ref, route_ref, probs_ref, th_ref, r_scratch,
                  *, lw, hw, n_cols):
    base = [qv_ref[c] * lw[c] + qv_ref[3 + c] * hw[c] for c in range(3)]
    raw = [base[c] + off_ref[c] for c in range(3)]
    t1 = raw[0]
    t2 = jnp.maximum(raw[1], t1 + MIN_GAP)
    t3 = jnp.maximum(raw[2], t2 + MIN_GAP)
    left_w = jnp.maximum(t2 - t1, 0.001)
    right_w = jnp.maximum(t3 - t2, 0.001)
    c0 = t1 - left_w
    c1 = (t1 + t2) * 0.5
    c2 = (t2 + t3) * 0.5
    c3 = t3 + right_w

    @pl.when(pl.program_id(0) == 0)
    def _():
        rows = jax.lax.broadcasted_iota(jnp.int32, (4 * n_cols, 4 * n_cols), 0)
        cols = jax.lax.broadcasted_iota(jnp.int32, (4 * n_cols, 4 * n_cols), 1)
        tgt = 4 * (rows % n_cols) + rows // n_cols
        r_scratch[...] = (cols == tgt).astype(jnp.float32)

    ed = ed_ref[...]
    d0 = jnp.abs(ed - c0)
    d1 = jnp.abs(ed - c1)
    d2 = jnp.abs(ed - c2)
    d3 = jnp.abs(ed - c3)
    l0 = -TEMPERATURE * d0
    l1 = -TEMPERATURE * d1
    l2 = -TEMPERATURE * d2
    l3 = -TEMPERATURE * d3
    m = jnp.maximum(jnp.maximum(l0, l1), jnp.maximum(l2, l3))
    e0 = jnp.exp(l0 - m)
    e1 = jnp.exp(l1 - m)
    e2 = jnp.exp(l2 - m)
    e3 = jnp.exp(l3 - m)
    ssum = (e0 + e1) + (e2 + e3)
    p = jnp.concatenate([e0 / ssum, e1 / ssum, e2 / ssum, e3 / ssum], axis=1)
    probs_ref[...] = jnp.dot(p, r_scratch[...], preferred_element_type=jnp.float32)

    i32 = jnp.int32
    route_ref[...] = ((ed > t1).astype(i32) + (ed > t2).astype(i32)
                      + (ed > t3).astype(i32))

    lane = jax.lax.broadcasted_iota(jnp.int32, (1, 8), 1)
    th = jnp.where(lane == 0, t1, jnp.where(lane == 1, t2, jnp.where(lane == 2, t3, 0.0)))
    th_ref[...] = th


def _routing_pass(ed, qvals, offs, lw, hw):
    n_rows, n_cols = ed.shape
    block = 512
    grid = (n_rows // block,)
    route, probs2d, th = pl.pallas_call(
        functools.partial(_routing_body, lw=tuple(np.float32(x) for x in lw),
                          hw=tuple(np.float32(x) for x in hw), n_cols=n_cols),
        grid=grid,
        in_specs=[
            pl.BlockSpec(memory_space=pltpu.SMEM),
            pl.BlockSpec(memory_space=pltpu.SMEM),
            pl.BlockSpec((block, n_cols), lambda i: (i, 0)),
        ],
        out_specs=[
            pl.BlockSpec((block, n_cols), lambda i: (i, 0)),
            pl.BlockSpec((block, 4 * n_cols), lambda i: (i, 0)),
            pl.BlockSpec((1, 8), lambda i: (0, 0)),
        ],
        out_shape=[
            jax.ShapeDtypeStruct((n_rows, n_cols), jnp.int32),
            jax.ShapeDtypeStruct((n_rows, 4 * n_cols), jnp.float32),
            jax.ShapeDtypeStruct((1, 8), jnp.float32),
        ],
        scratch_shapes=[pltpu.VMEM((4 * n_cols, 4 * n_cols), jnp.float32)],
    )(qvals, offs, ed)
    probs = probs2d.reshape(n_rows, n_cols, 4)
    thresholds = th.reshape(8)[:3]
    return route, probs, thresholds


def kernel(ed, threshold_offsets):
    n = ed.size
    low, high, lw, hw = _quantile_rank_weights(n)

    flat = ed.reshape(-1)
    qv16 = _sc_select(flat, tuple(int(x) for x in low))
    qvals = qv16[:8]

    offs = OFFSET_SCALE * jnp.tanh(threshold_offsets)
    offs = jnp.concatenate([offs, jnp.zeros((1,), jnp.float32)])

    route, probs, thresholds = _routing_pass(ed, qvals, offs, lw, hw)
    return route, probs, thresholds


# speedup vs baseline: 7.3209x; 1.1362x over previous
import functools

import numpy as np
import jax
import jax.numpy as jnp
from jax import lax
from jax.experimental import pallas as pl
from jax.experimental.pallas import tpu as pltpu
from jax.experimental.pallas import tpu_sc as plsc

TEMPERATURE = 8.0
OFFSET_SCALE = 0.2
MIN_GAP = 0.001

_I32MAX = np.int32(2147483647)
_MASK31 = np.int32(0x7FFFFFFF)


def _quantile_rank_weights(n: int):
    s = ((np.float32(0.55) + np.float32(0.25)) + np.float32(0.15)) + np.float32(0.05)
    fr = np.array([0.55, 0.25, 0.15, 0.05], dtype=np.float32) / np.maximum(s, np.float32(1e-8))
    cdf = np.cumsum(fr).astype(np.float32)[:3]
    q = (cdf * (np.float32(n) - np.float32(1.0))).astype(np.float32)
    low = np.clip(np.floor(q), 0, n - 1)
    high = np.clip(np.ceil(q), 0, n - 1)
    hw = (q - low).astype(np.float32)
    lw = (np.float32(1.0) - hw).astype(np.float32)
    return low.astype(np.int64), high.astype(np.int64), lw, hw


def _sortable_key(x):
    b = plsc.bitcast(x, jnp.int32)
    m = lax.shift_right_arithmetic(b, 31)
    return b ^ (m & _MASK31)


def _sc_select(ed_flat, ranks):
    n = ed_flat.shape[0]
    NT = 16
    per_tile = n // NT
    CH = 25600
    NCH = per_tile // CH
    assert per_tile % CH == 0 and CH % 16 == 0
    k0, k1, k2 = (jnp.int32(r) for r in ranks)

    mesh = plsc.VectorSubcoreMesh(core_axis_name="c", subcore_axis_name="s",
                                  num_cores=1)

    def merge_slabs(slab, acc, tmp, rows, halves):
        pltpu.sync_copy(slab.at[0], acc)

        def mbody(t, _):
            pltpu.sync_copy(slab.at[t], tmp)

            def rbody(rr, _):
                for hh in range(halves):
                    acc[rr, pl.ds(hh * 16, 16)] = (
                        acc[rr, pl.ds(hh * 16, 16)] + tmp[rr, pl.ds(hh * 16, 16)])
                return 0

            lax.fori_loop(0, rows, rbody, 0)
            return 0

        lax.fori_loop(1, NT, mbody, 0)

    def scan_hist(buf, rows, halves, kt):
        init = (jnp.zeros((16,), jnp.int32), jnp.zeros((16,), jnp.int32),
                jnp.zeros((16,), jnp.int32))

        def body(r, carry):
            cum, bc, bs = carry
            for h in range(halves):
                v = buf[r, pl.ds(h * 16, 16)]
                s = plsc.cumsum(v)
                cv = cum + s
                mlt = cv <= kt
                bc = bc + jnp.where(mlt, jnp.int32(1), jnp.int32(0))
                bs = bs + jnp.where(mlt, v, jnp.int32(0))
                cum = cum + jnp.max(s)
            return cum, bc, bs

        cum, bc, bs = lax.fori_loop(0, rows, body, init)
        return jnp.sum(bc), jnp.sum(bs)

    def zero2d(buf, rows, halves):
        z = jnp.zeros((16,), jnp.int32)

        def body(r, _):
            for h in range(halves):
                buf[r, pl.ds(h * 16, 16)] = z
            return 0

        lax.fori_loop(0, rows, body, 0)

    @functools.partial(
        pl.kernel,
        out_type=jax.ShapeDtypeStruct((16,), jnp.float32),
        mesh=mesh,
        compiler_params=pltpu.CompilerParams(needs_layout_passes=False),
        scratch_types=[
            pltpu.VMEM((2, CH), jnp.float32),
            pltpu.SemaphoreType.DMA,
            pltpu.SemaphoreType.DMA,
            pltpu.VMEM((32, 128), jnp.int32),
            pltpu.VMEM((32, 128), jnp.int32),
            pltpu.VMEM((32, 128), jnp.int32),
            pltpu.VMEM((32, 128), jnp.int32),
            pltpu.VMEM((2, 128), jnp.int32),
            pltpu.VMEM((2, 128), jnp.int32),
            pltpu.VMEM((2, 128), jnp.int32),
            pltpu.VMEM((32, 128), jnp.int32),
            pltpu.VMEM((2, 128), jnp.int32),
            pltpu.VMEM((16, 16), jnp.int32),
            pltpu.VMEM((3, 16), jnp.int32),
            pltpu.VMEM((16,), jnp.int32),
            pltpu.VMEM((16,), jnp.float32),
            pltpu.HBM((NT, 32, 128), jnp.int32),
            pltpu.HBM((NT, 32, 128), jnp.int32),
            pltpu.HBM((NT, 32, 128), jnp.int32),
            pltpu.HBM((NT, 32, 128), jnp.int32),
            pltpu.HBM((NT, 2, 128), jnp.int32),
            pltpu.HBM((NT, 2, 128), jnp.int32),
            pltpu.HBM((NT, 2, 128), jnp.int32),
            pltpu.HBM((16, 16), jnp.int32),
            pltpu.HBM((3, 16), jnp.int32),
        ],
    )
    def sel_kernel(ed_hbm, out_hbm, dbuf, sem0, sem1,
                   h1, h2_0, h2_1, h2_2, h3_0, h3_1, h3_2,
                   tmp, tmp3, minbuf, res_v, rowv, outv,
                   hb1, hb2_0, hb2_1, hb2_2, hb3_0, hb3_1, hb3_2, hbmin, hres):
        sid = lax.axis_index("s")
        base = sid * per_tile
        iota16 = lax.iota(jnp.int32, 16)
        h2s = (h2_0, h2_1, h2_2)
        hb2s = (hb2_0, hb2_1, hb2_2)
        h3s = (h3_0, h3_1, h3_2)
        hb3s = (hb3_0, hb3_1, hb3_2)
        ks = (k0, k1, k2)

        sems = (sem0, sem1)

        def sweep_data(elem_fn, carry=0):
            def start(c):
                return pltpu.async_copy(
                    ed_hbm.at[pl.ds(base + c * CH, CH)], dbuf.at[c % 2],
                    sems[c % 2])

            descs = [start(0), None]
            for c in range(NCH):
                if c + 1 < NCH:
                    descs[(c + 1) % 2] = start(c + 1)
                descs[c % 2].wait()
                bi = c % 2

                def elem_body(i, car, bi=bi):
                    x = dbuf[bi, pl.ds(i * 16, 16)]
                    return elem_fn(_sortable_key(x), car)

                carry = lax.fori_loop(0, CH // 16, elem_body, carry)
            return carry

        def publish(r, a, b):
            vec = jnp.where(iota16 == 0, a, jnp.where(iota16 == 1, b, jnp.int32(0)))
            rowv[...] = vec
            pltpu.sync_copy(rowv, hres.at[r])

        zero2d(h1, 32, 8)

        ones16 = jnp.full((16,), 1, jnp.int32)

        def p1(S, car):
            b = lax.shift_right_arithmetic(S, 20) + jnp.int32(2048)
            plsc.addupdate_scatter(
                h1, [lax.shift_right_logical(b, 7), b & jnp.int32(127)], ones16)
            return car

        sweep_data(p1)
        pltpu.sync_copy(h1, hb1.at[sid])
        plsc.subcore_barrier()
        for r in range(3):
            @pl.when(sid == r)
            def _(r=r):
                merge_slabs(hb1, h1, tmp, 32, 8)
                b, e = scan_hist(h1, 32, 8, ks[r])
                publish(r, b - jnp.int32(2048), ks[r] - e)
        plsc.subcore_barrier()
        pltpu.sync_copy(hres, res_v)
        bin1 = [res_v[r, :][0] for r in range(3)]
        g2 = [res_v[r, :][1] for r in range(3)]

        for r in range(3):
            zero2d(h2s[r], 32, 8)

        def p2(S, car):
            top = lax.shift_right_arithmetic(S, 20)
            mid = lax.shift_right_arithmetic(S, 8) & jnp.int32(0xFFF)
            row = lax.shift_right_logical(mid, 7)
            col = mid & jnp.int32(127)
            for r in range(3):
                match = top == bin1[r]
                plsc.addupdate_scatter(h2s[r], [row, col], ones16, mask=match)
            return car

        sweep_data(p2)
        for r in range(3):
            pltpu.sync_copy(h2s[r], hb2s[r].at[sid])
        plsc.subcore_barrier()
        for r in range(3):
            @pl.when(sid == r)
            def _(r=r):
                merge_slabs(hb2s[r], h2s[r], tmp, 32, 8)
                b, e = scan_hist(h2s[r], 32, 8, g2[r])
                publish(r, lax.shift_left(bin1[r], 12) | b, g2[r] - e)
        plsc.subcore_barrier()
        pltpu.sync_copy(hres, res_v)
        pref24 = [res_v[r, :][0] for r in range(3)]
        g3 = [res_v[r, :][1] for r in range(3)]

        for r in range(3):
            zero2d(h3s[r], 2, 8)

        def p3(S, car):
            top24 = lax.shift_right_arithmetic(S, 8)
            lowb = S & jnp.int32(0xFF)
            row = lax.shift_right_logical(lowb, 7)
            col = lowb & jnp.int32(127)
            n0, n1, n2 = car
            n0 = jnp.minimum(n0, jnp.where(top24 > pref24[0], S, _I32MAX))
            n1 = jnp.minimum(n1, jnp.where(top24 > pref24[1], S, _I32MAX))
            n2 = jnp.minimum(n2, jnp.where(top24 > pref24[2], S, _I32MAX))
            for r in range(3):
                match = top24 == pref24[r]
                plsc.addupdate_scatter(h3s[r], [row, col], ones16, mask=match)
            return n0, n1, n2

        init3 = tuple(jnp.full((16,), 2147483647, jnp.int32) for _ in range(3))
        n0, n1, n2 = sweep_data(p3, init3)
        mv = jnp.where(iota16 == 0, jnp.min(n0),
                       jnp.where(iota16 == 1, jnp.min(n1),
                                 jnp.where(iota16 == 2, jnp.min(n2), _I32MAX)))
        rowv[...] = mv
        pltpu.sync_copy(rowv, hbmin.at[sid])
        for r in range(3):
            pltpu.sync_copy(h3s[r], hb3s[r].at[sid])
        plsc.subcore_barrier()
        for r in range(3):
            @pl.when(sid == r)
            def _(r=r):
                merge_slabs(hb3s[r], h3s[r], tmp3, 2, 8)
                b, e = scan_hist(h3s[r], 2, 8, g3[r])
                g4 = g3[r] - e
                klo = lax.shift_left(pref24[r], 8) | b
                zeros16 = jnp.zeros((16,), jnp.int32)
                cnt_vec = plsc.load_gather(
                    h3s[r], [zeros16 + lax.shift_right_logical(b, 7),
                             zeros16 + (b & jnp.int32(127))])
                dup = g4 + jnp.int32(1) < jnp.max(cnt_vec)
                nb = jnp.full((16,), 256, jnp.int32)
                for rr in range(2):
                    for hh in range(8):
                        v = h3s[r][rr, pl.ds(hh * 16, 16)]
                        binidx = iota16 + jnp.int32(rr * 128 + hh * 16)
                        m = (v > jnp.int32(0)) & (binidx > b)
                        nb = jnp.minimum(nb, jnp.where(m, binidx, jnp.int32(256)))
                nextbin = jnp.min(nb)
                pltpu.sync_copy(hbmin, minbuf)
                acc = minbuf[0, :]
                for t in range(1, NT):
                    acc = jnp.minimum(acc, minbuf[t, :])
                khi = jnp.where(
                    dup, klo,
                    jnp.where(nextbin < jnp.int32(256),
                              lax.shift_left(pref24[r], 8) | nextbin, acc[r]))
                publish(r, klo, khi)
        plsc.subcore_barrier()
        pltpu.sync_copy(hres, res_v)

        @pl.when(sid == 0)
        def _():
            vec = jnp.zeros((16,), jnp.float32)
            for r in range(3):
                klo = res_v[r, :][0]
                khi = res_v[r, :][1]
                blo = jnp.where(klo >= 0, klo, klo ^ _MASK31)
                bhi = jnp.where(khi >= 0, khi, khi ^ _MASK31)
                vlo = lax.bitcast_convert_type(blo, jnp.float32)
                vhi = lax.bitcast_convert_type(bhi, jnp.float32)
                vec = jnp.where(iota16 == r, vlo, vec)
                vec = jnp.where(iota16 == r + 3, vhi, vec)
            outv[...] = vec
            pltpu.sync_copy(outv, out_hbm)

    return sel_kernel(ed_flat)


def _routing_body(qv_ref, off_ref, ed_ref, route_ref, probs_ref, th_ref, r_scratch,
                  *, lw, hw, n_cols):
    base = [qv_ref[c] * lw[c] + qv_ref[3 + c] * hw[c] for c in range(3)]
    raw = [base[c] + off_ref[c] for c in range(3)]
    t1 = raw[0]
    t2 = jnp.maximum(raw[1], t1 + MIN_GAP)
    t3 = jnp.maximum(raw[2], t2 + MIN_GAP)
    left_w = jnp.maximum(t2 - t1, 0.001)
    right_w = jnp.maximum(t3 - t2, 0.001)
    c0 = t1 - left_w
    c1 = (t1 + t2) * 0.5
    c2 = (t2 + t3) * 0.5
    c3 = t3 + right_w

    @pl.when(pl.program_id(0) == 0)
    def _():
        rows = jax.lax.broadcasted_iota(jnp.int32, (4 * n_cols, 4 * n_cols), 0)
        cols = jax.lax.broadcasted_iota(jnp.int32, (4 * n_cols, 4 * n_cols), 1)
        tgt = 4 * (rows % n_cols) + rows // n_cols
        r_scratch[...] = (cols == tgt).astype(jnp.float32)

    ed = ed_ref[...]
    d0 = jnp.abs(ed - c0)
    d1 = jnp.abs(ed - c1)
    d2 = jnp.abs(ed - c2)
    d3 = jnp.abs(ed - c3)
    l0 = -TEMPERATURE * d0
    l1 = -TEMPERATURE * d1
    l2 = -TEMPERATURE * d2
    l3 = -TEMPERATURE * d3
    m = jnp.maximum(jnp.maximum(l0, l1), jnp.maximum(l2, l3))
    e0 = jnp.exp(l0 - m)
    e1 = jnp.exp(l1 - m)
    e2 = jnp.exp(l2 - m)
    e3 = jnp.exp(l3 - m)
    ssum = (e0 + e1) + (e2 + e3)
    p = jnp.concatenate([e0 / ssum, e1 / ssum, e2 / ssum, e3 / ssum], axis=1)
    probs_ref[...] = jnp.dot(p, r_scratch[...], preferred_element_type=jnp.float32)

    i32 = jnp.int32
    route_ref[...] = ((ed > t1).astype(i32) + (ed > t2).astype(i32)
                      + (ed > t3).astype(i32))

    lane = jax.lax.broadcasted_iota(jnp.int32, (1, 8), 1)
    th = jnp.where(lane == 0, t1, jnp.where(lane == 1, t2, jnp.where(lane == 2, t3, 0.0)))
    th_ref[...] = th


def _routing_pass(ed, qvals, offs, lw, hw):
    n_rows, n_cols = ed.shape
    block = 512
    grid = (n_rows // block,)
    route, probs2d, th = pl.pallas_call(
        functools.partial(_routing_body, lw=tuple(np.float32(x) for x in lw),
                          hw=tuple(np.float32(x) for x in hw), n_cols=n_cols),
        grid=grid,
        in_specs=[
            pl.BlockSpec(memory_space=pltpu.SMEM),
            pl.BlockSpec(memory_space=pltpu.SMEM),
            pl.BlockSpec((block, n_cols), lambda i: (i, 0)),
        ],
        out_specs=[
            pl.BlockSpec((block, n_cols), lambda i: (i, 0)),
            pl.BlockSpec((block, 4 * n_cols), lambda i: (i, 0)),
            pl.BlockSpec((1, 8), lambda i: (0, 0)),
        ],
        out_shape=[
            jax.ShapeDtypeStruct((n_rows, n_cols), jnp.int32),
            jax.ShapeDtypeStruct((n_rows, 4 * n_cols), jnp.float32),
            jax.ShapeDtypeStruct((1, 8), jnp.float32),
        ],
        scratch_shapes=[pltpu.VMEM((4 * n_cols, 4 * n_cols), jnp.float32)],
    )(qvals, offs, ed)
    probs = probs2d.reshape(n_rows, n_cols, 4)
    thresholds = th.reshape(8)[:3]
    return route, probs, thresholds


def kernel(ed, threshold_offsets):
    n = ed.size
    low, high, lw, hw = _quantile_rank_weights(n)

    flat = ed.reshape(-1)
    qv16 = _sc_select(flat, tuple(int(x) for x in low))
    qvals = qv16[:8]

    offs = OFFSET_SCALE * jnp.tanh(threshold_offsets)
    offs = jnp.concatenate([offs, jnp.zeros((1,), jnp.float32)])

    route, probs, thresholds = _routing_pass(ed, qvals, offs, lw, hw)
    return route, probs, thresholds
